# Initial kernel scaffold; baseline (speedup 1.0000x reference)
import jax
import jax.numpy as jnp
from jax.experimental import pallas as pl


def kernel(x, dn0_w1, dn0_b1, dn0_w2, dn0_b2, dn1_w1, dn1_b1, dn1_w2, dn1_b2, dn2_w1, dn2_b1, dn2_w2, dn2_b2, dn3_w1, dn3_b1, dn3_w2, dn3_b2, dn4_w1, dn4_b1, dn4_w2, dn4_b2, dn5_w1, dn5_b1, dn5_w2, dn5_b2, lin_w, lin_b, h_w0, h_b0, h_w1, h_b1, h_w2, h_b2, h_out_w, h_out_b, w_w0, w_b0, w_w1, w_b1, w_w2, w_b2, w_out_w, w_out_b):
    raise NotImplementedError("write your pallas kernel here")


# speedup vs baseline: 12.9301x; 12.9301x over previous
import jax
import jax.numpy as jnp
from jax import lax
from jax.experimental import pallas as pl
from jax.experimental.pallas import tpu as pltpu

_GS = [33, 17, 9, 5, 3, 2]
_CIN = [1, 4, 8, 16, 32, 64]
_COUT = [4, 8, 16, 32, 64, 64]
_NB = 128


def _build_pq(pq, src, row_stride, G):
    M = G * G
    cnt = G - 1
    for p in (0, 1):
        for q in (0, 1):
            ph = 2 * p + q
            if p == 0:
                pq[ph, :, 0:G, :] = jnp.zeros_like(pq[ph, :, 0:G, :])
            else:
                pq[ph, :, (G - 1) * G:M, :] = jnp.zeros_like(
                    pq[ph, :, (G - 1) * G:M, :])
            if q == 0:
                pq[ph, :, 0:M:G, :] = jnp.zeros_like(pq[ph, :, 0:M:G, :])
            else:
                pq[ph, :, G - 1:M:G, :] = jnp.zeros_like(
                    pq[ph, :, G - 1:M:G, :])
            pq[ph, :, M:M + G + 1, :] = jnp.zeros_like(
                pq[ph, :, M:M + G + 1, :])
            gys = range(1, G) if p == 0 else range(0, G - 1)
            for gy in gys:
                ry = 2 * gy + p - 1
                base = ry * row_stride
                if q == 1:
                    s0, d0 = base, gy * G
                else:
                    s0, d0 = base + 1, gy * G + 1
                pq[ph, :, d0:d0 + cnt, :] = src[:, s0:s0 + 2 * cnt - 1:2, :]


def _fused_kernel(xph_ref,
                  w10, b10, w20, b20, w11, b11, w21, b21,
                  w12, b12, w22, b22, w13, b13, w23, b23,
                  w14, b14, w24, b24, w15, b15, w25, b25,
                  lw, lb, hw0, hb0, hw1, hb1, hw2, hb2, how, hob,
                  ww0, wb0, ww1, wb1, ww2, wb2, wow, wob,
                  o_ref,
                  ms0, os0, pq1, ms1, os1, pq2, ms2, os2,
                  pq3, ms3, os3, pq4, ms4, os4, pq5, ms5):
    w1s = [w10, w11, w12, w13, w14, w15]
    b1s = [b10, b11, b12, b13, b14, b15]
    w2s = [w20, w21, w22, w23, w24, w25]
    b2s = [b20, b21, b22, b23, b24, b25]
    pqs = [None, pq1, pq2, pq3, pq4, pq5]
    mss = [ms0, ms1, ms2, ms3, ms4, ms5]
    oss = [os0, os1, os2, os3, os4, None]
    dn = (((1,), (0,)), ((), ()))

    out = None
    for i in range(6):
        G, cin, cout = _GS[i], _CIN[i], _COUT[i]
        M, P = G * G, G + 1
        pq, ms = pqs[i], mss[i]

        acc = jnp.zeros((cout, M, _NB), jnp.float32)
        for dy in range(3):
            for dx in range(3):
                ph = 2 * (dy % 2) + (dx % 2)
                sh = (dy // 2) * G + (dx // 2)
                if i == 0:
                    xs = xph_ref[ph, 0, sh:sh + M, :]
                    wt = w1s[0][3 * dy + dx]
                    acc = acc + wt[:, None, None] * xs[None]
                else:
                    xs = pq[ph, :, sh:sh + M, :]
                    acc = acc + lax.dot_general(
                        w1s[i][3 * dy + dx], xs, dn,
                        preferred_element_type=jnp.float32)
        m_idx = lax.broadcasted_iota(jnp.int32, (M, _NB), 0)
        mask = jnp.where((m_idx // G < G - 1) & (m_idx % G < G - 1), 1.0, 0.0)
        mid = jnp.maximum(acc + b1s[i][...][:, :, None], 0.0) * mask[None]

        ms[:, 0:P, :] = jnp.zeros_like(ms[:, 0:P, :])
        ms[:, P + M:P + M + P, :] = jnp.zeros_like(ms[:, P + M:P + M + P, :])
        ms[:, P:P + M, :] = mid
        acc2 = jnp.zeros((cout, M, _NB), jnp.float32)
        for dy in range(3):
            for dx in range(3):
                sh = (dy - 1) * G + (dx - 1)
                acc2 = acc2 + lax.dot_general(
                    w2s[i][3 * dy + dx], ms[:, P + sh:P + sh + M, :], dn,
                    preferred_element_type=jnp.float32)
        out = jnp.maximum(acc2 + b2s[i][...][:, :, None], 0.0)

        if i < 5:
            oss[i][...] = out
            _build_pq(pqs[i + 1], oss[i], G, _GS[i + 1])

    feat = jnp.transpose(out[:, 0, :])
    z = jnp.dot(feat, lw[...], preferred_element_type=jnp.float32) + lb[...]
    h = z
    for wr, br in ((hw0, hb0), (hw1, hb1), (hw2, hb2)):
        h = jnp.maximum(
            jnp.dot(h, wr[...], preferred_element_type=jnp.float32) + br[...],
            0.0)
    h = jnp.tanh(
        jnp.dot(h, how[...], preferred_element_type=jnp.float32) + hob[...])
    w = z
    for wr, br in ((ww0, wb0), (ww1, wb1), (ww2, wb2)):
        w = jnp.maximum(
            jnp.dot(w, wr[...], preferred_element_type=jnp.float32) + br[...],
            0.0)
    w = jnp.tanh(
        jnp.dot(w, wow[...], preferred_element_type=jnp.float32) + wob[...])
    pat = h[:, :, None] + w[:, None, :]
    o_ref[...] = jnp.clip(pat, -1.0, 1.0) * 2.0 - 1.0


def kernel(x, dn0_w1, dn0_b1, dn0_w2, dn0_b2, dn1_w1, dn1_b1, dn1_w2, dn1_b2,
           dn2_w1, dn2_b1, dn2_w2, dn2_b2, dn3_w1, dn3_b1, dn3_w2, dn3_b2,
           dn4_w1, dn4_b1, dn4_w2, dn4_b2, dn5_w1, dn5_b1, dn5_w2, dn5_b2,
           lin_w, lin_b, h_w0, h_b0, h_w1, h_b1, h_w2, h_b2,
           h_out_w, h_out_b, w_w0, w_b0, w_w1, w_b1, w_w2, w_b2,
           w_out_w, w_out_b):
    N = x.shape[0]
    xp = jnp.pad(x.reshape(N, 64, 64).astype(jnp.float32),
                 ((0, 0), (1, 1), (1, 1)))
    xp = xp.reshape(N, 33, 2, 33, 2).transpose(2, 4, 1, 3, 0)
    xph = jnp.pad(xp.reshape(4, 33 * 33, N),
                  ((0, 0), (0, 34), (0, 0))).reshape(4, 1, 1123, N)

    args = (xph,
            dn0_w1.reshape(9, 4), dn0_b1, dn0_w2, dn0_b2,
            dn1_w1, dn1_b1, dn1_w2, dn1_b2,
            dn2_w1, dn2_b1, dn2_w2, dn2_b2,
            dn3_w1, dn3_b1, dn3_w2, dn3_b2,
            dn4_w1, dn4_b1, dn4_w2, dn4_b2,
            dn5_w1, dn5_b1, dn5_w2, dn5_b2,
            lin_w, lin_b, h_w0, h_b0, h_w1, h_b1, h_w2, h_b2,
            h_out_w, h_out_b, w_w0, w_b0, w_w1, w_b1, w_w2, w_b2,
            w_out_w, w_out_b)

    in_specs = [pl.BlockSpec((4, 1, 1123, _NB), lambda n: (0, 0, 0, n))]
    for a in args[1:]:
        r = a.ndim
        in_specs.append(pl.BlockSpec(a.shape, lambda n, _r=r: (0,) * _r))

    scratch = []
    for i in range(6):
        G, cin, cout = _GS[i], _CIN[i], _COUT[i]
        M = G * G
        if i > 0:
            scratch.append(pltpu.VMEM((4, cin, M + G + 1, _NB), jnp.float32))
        scratch.append(pltpu.VMEM((cout, M + 2 * (G + 1), _NB), jnp.float32))
        if i < 5:
            scratch.append(pltpu.VMEM((cout, M, _NB), jnp.float32))

    pat = pl.pallas_call(
        _fused_kernel,
        out_shape=jax.ShapeDtypeStruct((N, 16, 128), jnp.float32),
        grid=(N // _NB,),
        in_specs=in_specs,
        out_specs=pl.BlockSpec((_NB, 16, 128), lambda n: (n, 0, 0)),
        scratch_shapes=scratch,
        compiler_params=pltpu.CompilerParams(
            dimension_semantics=("parallel",),
            vmem_limit_bytes=60 * 1024 * 1024),
    )(*args)
    return pat[:, None, :, :]


# speedup vs baseline: 18.5344x; 1.4334x over previous
import jax
import jax.numpy as jnp
from jax import lax
from jax.experimental import pallas as pl
from jax.experimental.pallas import tpu as pltpu

_GS = [33, 17, 9, 5, 3, 2]
_CIN = [1, 4, 8, 16, 32, 64]
_COUT = [4, 8, 16, 32, 64, 64]
_NB = 128


def _build_pq(pq, src, row_stride, G, src_is_2d=False):
    M = G * G
    cnt = G - 1
    for p in (0, 1):
        for q in (0, 1):
            ph = 2 * p + q
            if p == 0:
                pq[ph, :, 0:G, :] = jnp.zeros_like(pq[ph, :, 0:G, :])
            else:
                pq[ph, :, (G - 1) * G:M, :] = jnp.zeros_like(
                    pq[ph, :, (G - 1) * G:M, :])
            if q == 0:
                pq[ph, :, 0:M:G, :] = jnp.zeros_like(pq[ph, :, 0:M:G, :])
            else:
                pq[ph, :, G - 1:M:G, :] = jnp.zeros_like(
                    pq[ph, :, G - 1:M:G, :])
            pq[ph, :, M:M + G + 1, :] = jnp.zeros_like(
                pq[ph, :, M:M + G + 1, :])
            gys = range(1, G) if p == 0 else range(0, G - 1)
            for gy in gys:
                ry = 2 * gy + p - 1
                base = ry * row_stride
                if q == 1:
                    s0, d0 = base, gy * G
                else:
                    s0, d0 = base + 1, gy * G + 1
                if src_is_2d:
                    pq[ph, 0, d0:d0 + cnt, :] = src[s0:s0 + 2 * cnt - 1:2, :]
                else:
                    pq[ph, :, d0:d0 + cnt, :] = src[:, s0:s0 + 2 * cnt - 1:2, :]


def _fused_kernel(x_ref,
                  w10, b10, w20, b20, w11, b11, w21, b21,
                  w12, b12, w22, b22, w13, b13, w23, b23,
                  w14, b14, w24, b24, w15, b15, w25, b25,
                  lw, lb, hw0, hb0, hw1, hb1, hw2, hb2, how, hob,
                  ww0, wb0, ww1, wb1, ww2, wb2, wow, wob,
                  o_ref,
                  xs_, pq0, ms0, os0, pq1, ms1, os1, pq2, ms2, os2,
                  pq3, ms3, os3, pq4, ms4, os4, pq5, ms5):
    w1s = [w10, w11, w12, w13, w14, w15]
    b1s = [b10, b11, b12, b13, b14, b15]
    w2s = [w20, w21, w22, w23, w24, w25]
    b2s = [b20, b21, b22, b23, b24, b25]
    pqs = [pq0, pq1, pq2, pq3, pq4, pq5]
    mss = [ms0, ms1, ms2, ms3, ms4, ms5]
    oss = [os0, os1, os2, os3, os4, None]
    dn = (((1,), (0,)), ((), ()))

    xs_[...] = jnp.transpose(x_ref[...])
    _build_pq(pq0, xs_, 64, _GS[0], True)

    out = None
    for i in range(6):
        G, cin, cout = _GS[i], _CIN[i], _COUT[i]
        M, P = G * G, G + 1
        pq, ms = pqs[i], mss[i]

        acc = jnp.zeros((cout, M, _NB), jnp.float32)
        for dy in range(3):
            for dx in range(3):
                ph = 2 * (dy % 2) + (dx % 2)
                sh = (dy // 2) * G + (dx // 2)
                if i == 0:
                    xs = pq[ph, 0, sh:sh + M, :]
                    wt = w1s[0][3 * dy + dx]
                    acc = acc + wt[:, None, None] * xs[None]
                else:
                    xs = pq[ph, :, sh:sh + M, :]
                    acc = acc + lax.dot_general(
                        w1s[i][3 * dy + dx], xs, dn,
                        preferred_element_type=jnp.float32)
        m_idx = lax.broadcasted_iota(jnp.int32, (M, _NB), 0)
        mask = jnp.where((m_idx // G < G - 1) & (m_idx % G < G - 1), 1.0, 0.0)
        mid = jnp.maximum(acc + b1s[i][...][:, :, None], 0.0) * mask[None]

        ms[:, 0:P, :] = jnp.zeros_like(ms[:, 0:P, :])
        ms[:, P + M:P + M + P, :] = jnp.zeros_like(ms[:, P + M:P + M + P, :])
        ms[:, P:P + M, :] = mid
        acc2 = jnp.zeros((cout, M, _NB), jnp.float32)
        for dy in range(3):
            for dx in range(3):
                sh = (dy - 1) * G + (dx - 1)
                acc2 = acc2 + lax.dot_general(
                    w2s[i][3 * dy + dx], ms[:, P + sh:P + sh + M, :], dn,
                    preferred_element_type=jnp.float32)
        out = jnp.maximum(acc2 + b2s[i][...][:, :, None], 0.0)

        if i < 5:
            oss[i][...] = out
            _build_pq(pqs[i + 1], oss[i], G, _GS[i + 1])

    feat = jnp.transpose(out[:, 0, :])
    z = jnp.dot(feat, lw[...], preferred_element_type=jnp.float32) + lb[...]
    h = z
    for wr, br in ((hw0, hb0), (hw1, hb1), (hw2, hb2)):
        h = jnp.maximum(
            jnp.dot(h, wr[...], preferred_element_type=jnp.float32) + br[...],
            0.0)
    h = jnp.tanh(
        jnp.dot(h, how[...], preferred_element_type=jnp.float32) + hob[...])
    w = z
    for wr, br in ((ww0, wb0), (ww1, wb1), (ww2, wb2)):
        w = jnp.maximum(
            jnp.dot(w, wr[...], preferred_element_type=jnp.float32) + br[...],
            0.0)
    w = jnp.tanh(
        jnp.dot(w, wow[...], preferred_element_type=jnp.float32) + wob[...])
    pat = h[:, :, None] + w[:, None, :]
    o_ref[...] = jnp.clip(pat, -1.0, 1.0) * 2.0 - 1.0


def kernel(x, dn0_w1, dn0_b1, dn0_w2, dn0_b2, dn1_w1, dn1_b1, dn1_w2, dn1_b2,
           dn2_w1, dn2_b1, dn2_w2, dn2_b2, dn3_w1, dn3_b1, dn3_w2, dn3_b2,
           dn4_w1, dn4_b1, dn4_w2, dn4_b2, dn5_w1, dn5_b1, dn5_w2, dn5_b2,
           lin_w, lin_b, h_w0, h_b0, h_w1, h_b1, h_w2, h_b2,
           h_out_w, h_out_b, w_w0, w_b0, w_w1, w_b1, w_w2, w_b2,
           w_out_w, w_out_b):
    N = x.shape[0]
    x2 = x.reshape(N, 64 * 64).astype(jnp.float32)

    args = (x2,
            dn0_w1.reshape(9, 4), dn0_b1, dn0_w2, dn0_b2,
            dn1_w1, dn1_b1, dn1_w2, dn1_b2,
            dn2_w1, dn2_b1, dn2_w2, dn2_b2,
            dn3_w1, dn3_b1, dn3_w2, dn3_b2,
            dn4_w1, dn4_b1, dn4_w2, dn4_b2,
            dn5_w1, dn5_b1, dn5_w2, dn5_b2,
            lin_w, lin_b, h_w0, h_b0, h_w1, h_b1, h_w2, h_b2,
            h_out_w, h_out_b, w_w0, w_b0, w_w1, w_b1, w_w2, w_b2,
            w_out_w, w_out_b)

    in_specs = [pl.BlockSpec((_NB, 64 * 64), lambda n: (n, 0))]
    for a in args[1:]:
        r = a.ndim
        in_specs.append(pl.BlockSpec(a.shape, lambda n, _r=r: (0,) * _r))

    scratch = [pltpu.VMEM((64 * 64, _NB), jnp.float32)]
    for i in range(6):
        G, cin, cout = _GS[i], _CIN[i], _COUT[i]
        M = G * G
        scratch.append(pltpu.VMEM((4, cin, M + G + 1, _NB), jnp.float32))
        scratch.append(pltpu.VMEM((cout, M + 2 * (G + 1), _NB), jnp.float32))
        if i < 5:
            scratch.append(pltpu.VMEM((cout, M, _NB), jnp.float32))

    pat = pl.pallas_call(
        _fused_kernel,
        out_shape=jax.ShapeDtypeStruct((N, 16, 128), jnp.float32),
        grid=(N // _NB,),
        in_specs=in_specs,
        out_specs=pl.BlockSpec((_NB, 16, 128), lambda n: (n, 0, 0)),
        scratch_shapes=scratch,
        compiler_params=pltpu.CompilerParams(
            dimension_semantics=("parallel",),
            vmem_limit_bytes=60 * 1024 * 1024),
    )(*args)
    return pat[:, None, :, :]


# speedup vs baseline: 25.8174x; 1.3929x over previous
import numpy as _np
import jax
import jax.numpy as jnp
from jax import lax
from jax.experimental import pallas as pl
from jax.experimental.pallas import tpu as pltpu

_GS = [33, 17, 9, 5, 3, 2]
_CIN = [1, 4, 8, 16, 32, 64]
_COUT = [4, 8, 16, 32, 64, 64]
_NB = 128


def _build_pq(pq, src, row_stride, G, src_is_2d=False):
    M = G * G
    cnt = G - 1
    for p in (0, 1):
        for q in (0, 1):
            ph = 2 * p + q
            if p == 0:
                pq[ph, :, 0:G, :] = jnp.zeros_like(pq[ph, :, 0:G, :])
            else:
                pq[ph, :, (G - 1) * G:M, :] = jnp.zeros_like(
                    pq[ph, :, (G - 1) * G:M, :])
            if q == 0:
                pq[ph, :, 0:M:G, :] = jnp.zeros_like(pq[ph, :, 0:M:G, :])
            else:
                pq[ph, :, G - 1:M:G, :] = jnp.zeros_like(
                    pq[ph, :, G - 1:M:G, :])
            pq[ph, :, M:M + G + 1, :] = jnp.zeros_like(
                pq[ph, :, M:M + G + 1, :])
            gys = range(1, G) if p == 0 else range(0, G - 1)
            for gy in gys:
                ry = 2 * gy + p - 1
                base = ry * row_stride
                if q == 1:
                    s0, d0 = base, gy * G
                else:
                    s0, d0 = base + 1, gy * G + 1
                if src_is_2d:
                    pq[ph, 0, d0:d0 + cnt, :] = src[s0:s0 + 2 * cnt - 1:2, :]
                else:
                    pq[ph, :, d0:d0 + cnt, :] = src[:, s0:s0 + 2 * cnt - 1:2, :]


def _fused_kernel(x_ref,
                  w10, b10, w20, b20, w11, b11, w21, b21,
                  w12, b12, w22, b22, w13, b13, w23, b23,
                  w14, b14, w24, b24, w15, b15, w25, b25,
                  lw, lb, hw0, hb0, hw1, hb1, hw2, hb2, how, hob,
                  ww0, wb0, ww1, wb1, ww2, wb2, wow, wob,
                  mk0, mk1, mk2, mk3, mk4, mk5,
                  o_ref,
                  xs_, pq0, pf0, ms0, os0, pq1, pf1, ms1, os1,
                  pq2, pf2, ms2, os2, pq3, pf3, ms3, os3,
                  pq4, pf4, ms4, os4, pq5, pf5, ms5):
    w1s = [w10, w11, w12, w13, w14, w15]
    b1s = [b10, b11, b12, b13, b14, b15]
    w2s = [w20, w21, w22, w23, w24, w25]
    b2s = [b20, b21, b22, b23, b24, b25]
    pqs = [pq0, pq1, pq2, pq3, pq4, pq5]
    pfs = [pf0, pf1, pf2, pf3, pf4, pf5]
    mss = [ms0, ms1, ms2, ms3, ms4, ms5]
    oss = [os0, os1, os2, os3, os4, None]
    mks = [mk0, mk1, mk2, mk3, mk4, mk5]
    B = _NB

    xs_[...] = jnp.transpose(x_ref[...])
    _build_pq(pq0, xs_, 64, _GS[0], True)

    out = None
    for i in range(6):
        G, cin, cout = _GS[i], _CIN[i], _COUT[i]
        M, P = G * G, G + 1
        pq, pf, ms = pqs[i], pfs[i], mss[i]

        Mw = M + G + 1
        for ph in range(4):
            pf[ph] = pq[ph].reshape(cin, Mw * B)

        acc = jnp.zeros((cout, M * B), jnp.float32)
        for dy in range(3):
            for dx in range(3):
                ph = 2 * (dy % 2) + (dx % 2)
                sh = ((dy // 2) * G + (dx // 2)) * B
                xs = pf[ph, :, sh:sh + M * B]
                if i == 0:
                    wt = w1s[0][3 * dy + dx]
                    acc = acc + wt[:, None] * xs
                else:
                    acc = acc + jnp.dot(w1s[i][3 * dy + dx], xs,
                                        preferred_element_type=jnp.float32)
        mid = jnp.maximum(acc + b1s[i][...], 0.0) * mks[i][...]

        ms[:, 0:P * B] = jnp.zeros_like(ms[:, 0:P * B])
        ms[:, (P + M) * B:] = jnp.zeros_like(ms[:, (P + M) * B:])
        ms[:, P * B:(P + M) * B] = mid
        acc2 = jnp.zeros((cout, M * B), jnp.float32)
        for dy in range(3):
            for dx in range(3):
                sh = ((dy - 1) * G + (dx - 1)) * B
                acc2 = acc2 + jnp.dot(w2s[i][3 * dy + dx],
                                      ms[:, P * B + sh:P * B + sh + M * B],
                                      preferred_element_type=jnp.float32)
        out = jnp.maximum(acc2 + b2s[i][...], 0.0)

        if i < 5:
            oss[i][...] = out.reshape(cout, M, B)
            _build_pq(pqs[i + 1], oss[i], G, _GS[i + 1])

    feat = jnp.transpose(out[:, 0:_NB])
    z = jnp.dot(feat, lw[...], preferred_element_type=jnp.float32) + lb[...]
    h = z
    for wr, br in ((hw0, hb0), (hw1, hb1), (hw2, hb2)):
        h = jnp.maximum(
            jnp.dot(h, wr[...], preferred_element_type=jnp.float32) + br[...],
            0.0)
    h = jnp.tanh(
        jnp.dot(h, how[...], preferred_element_type=jnp.float32) + hob[...])
    w = z
    for wr, br in ((ww0, wb0), (ww1, wb1), (ww2, wb2)):
        w = jnp.maximum(
            jnp.dot(w, wr[...], preferred_element_type=jnp.float32) + br[...],
            0.0)
    w = jnp.tanh(
        jnp.dot(w, wow[...], preferred_element_type=jnp.float32) + wob[...])
    pat = h[:, :, None] + w[:, None, :]
    o_ref[...] = jnp.clip(pat, -1.0, 1.0) * 2.0 - 1.0


def kernel(x, dn0_w1, dn0_b1, dn0_w2, dn0_b2, dn1_w1, dn1_b1, dn1_w2, dn1_b2,
           dn2_w1, dn2_b1, dn2_w2, dn2_b2, dn3_w1, dn3_b1, dn3_w2, dn3_b2,
           dn4_w1, dn4_b1, dn4_w2, dn4_b2, dn5_w1, dn5_b1, dn5_w2, dn5_b2,
           lin_w, lin_b, h_w0, h_b0, h_w1, h_b1, h_w2, h_b2,
           h_out_w, h_out_b, w_w0, w_b0, w_w1, w_b1, w_w2, w_b2,
           w_out_w, w_out_b):
    N = x.shape[0]
    x2 = x.reshape(N, 64 * 64).astype(jnp.float32)

    masks = []
    for G in _GS:
        m = _np.arange(G * G)
        v = ((m // G < G - 1) & (m % G < G - 1)).astype(_np.float32)
        masks.append(jnp.asarray(_np.repeat(v, _NB)[None, :]))

    args = (x2,
            dn0_w1.reshape(9, 4), dn0_b1, dn0_w2, dn0_b2,
            dn1_w1, dn1_b1, dn1_w2, dn1_b2,
            dn2_w1, dn2_b1, dn2_w2, dn2_b2,
            dn3_w1, dn3_b1, dn3_w2, dn3_b2,
            dn4_w1, dn4_b1, dn4_w2, dn4_b2,
            dn5_w1, dn5_b1, dn5_w2, dn5_b2,
            lin_w, lin_b, h_w0, h_b0, h_w1, h_b1, h_w2, h_b2,
            h_out_w, h_out_b, w_w0, w_b0, w_w1, w_b1, w_w2, w_b2,
            w_out_w, w_out_b) + tuple(masks)

    in_specs = [pl.BlockSpec((_NB, 64 * 64), lambda n: (n, 0))]
    for a in args[1:]:
        r = a.ndim
        in_specs.append(pl.BlockSpec(a.shape, lambda n, _r=r: (0,) * _r))

    scratch = [pltpu.VMEM((64 * 64, _NB), jnp.float32)]
    for i in range(6):
        G, cin, cout = _GS[i], _CIN[i], _COUT[i]
        M = G * G
        scratch.append(pltpu.VMEM((4, cin, M + G + 1, _NB), jnp.float32))
        scratch.append(pltpu.VMEM((4, cin, (M + G + 1) * _NB), jnp.float32))
        scratch.append(pltpu.VMEM((cout, (M + 2 * (G + 1)) * _NB), jnp.float32))
        if i < 5:
            scratch.append(pltpu.VMEM((cout, M, _NB), jnp.float32))

    pat = pl.pallas_call(
        _fused_kernel,
        out_shape=jax.ShapeDtypeStruct((N, 16, 128), jnp.float32),
        grid=(N // _NB,),
        in_specs=in_specs,
        out_specs=pl.BlockSpec((_NB, 16, 128), lambda n: (n, 0, 0)),
        scratch_shapes=scratch,
        compiler_params=pltpu.CompilerParams(
            dimension_semantics=("parallel",),
            vmem_limit_bytes=60 * 1024 * 1024),
    )(*args)
    return pat[:, None, :, :]
